# Initial kernel scaffold; baseline (speedup 1.0000x reference)
import jax
import jax.numpy as jnp
from jax.experimental import pallas as pl


def kernel(predicted_locs, predicted_scores, boxes, priors_cxcy):
    raise NotImplementedError("write your pallas kernel here")


# speedup vs baseline: 4.6467x; 4.6467x over previous
import functools

import jax
import jax.numpy as jnp
from jax.experimental import pallas as pl

_THRESHOLD = 0.2
_NEG_POS_RATIO = 3.0
_ALPHA = 1.0
_BISECT_ITERS = 30


def _mbl_kernel(locs_ref, scores_ref, boxes_ref, priors_ref, out_ref, *,
                p_real, n_obj, n_cls, p_pad):
    f32 = jnp.float32
    lane = jax.lax.broadcasted_iota(jnp.int32, (1, p_pad), 1)
    valid = lane < p_real

    pri = priors_ref[...]
    pcx = pri[0:1, :]
    pcy = pri[1:2, :]
    pw = pri[2:3, :]
    ph = pri[3:4, :]
    px0 = pcx - pw / 2.0
    py0 = pcy - ph / 2.0
    px1 = pcx + pw / 2.0
    py1 = pcy + ph / 2.0
    a2 = (px1 - px0) * (py1 - py0)

    boxes = boxes_ref[0]

    m = jnp.full((1, p_pad), -1.0, f32)
    mx0 = jnp.zeros((1, p_pad), f32)
    my0 = jnp.zeros((1, p_pad), f32)
    mx1 = jnp.zeros((1, p_pad), f32)
    my1 = jnp.zeros((1, p_pad), f32)
    best_prior = []
    for j in range(n_obj):
        bx0 = boxes[j:j + 1, 0:1]
        by0 = boxes[j:j + 1, 1:2]
        bx1 = boxes[j:j + 1, 2:3]
        by1 = boxes[j:j + 1, 3:4]
        lox = jnp.maximum(bx0, px0)
        loy = jnp.maximum(by0, py0)
        hix = jnp.minimum(bx1, px1)
        hiy = jnp.minimum(by1, py1)
        wx = jnp.clip(hix - lox, 0.0, None)
        wy = jnp.clip(hiy - loy, 0.0, None)
        inter = wx * wy
        a1 = (bx1 - bx0) * (by1 - by0)
        union = a1 + a2 - inter
        ov = inter / union
        mj = jnp.max(ov, axis=1, keepdims=True)
        pj = jnp.min(jnp.where(ov == mj, lane, p_pad), axis=1, keepdims=True)
        best_prior.append(pj)
        upd = ov > m
        m = jnp.where(upd, ov, m)
        mx0 = jnp.where(upd, bx0, mx0)
        my0 = jnp.where(upd, by0, my0)
        mx1 = jnp.where(upd, bx1, mx1)
        my1 = jnp.where(upd, by1, my1)

    for j in range(n_obj):
        cond = lane == best_prior[j]
        m = jnp.where(cond, 1.0, m)
        mx0 = jnp.where(cond, boxes[j:j + 1, 0:1], mx0)
        my0 = jnp.where(cond, boxes[j:j + 1, 1:2], my0)
        mx1 = jnp.where(cond, boxes[j:j + 1, 2:3], mx1)
        my1 = jnp.where(cond, boxes[j:j + 1, 3:4], my1)

    pos = jnp.logical_not(m < _THRESHOLD)

    cx = (mx0 + mx1) / 2.0
    cy = (my0 + my1) / 2.0
    w = mx1 - mx0
    h = my1 - my0
    gx = (cx - pcx) / (pw / 10.0)
    gy = (cy - pcy) / (ph / 10.0)
    gw = jnp.log(w / pw) * 5.0
    gh = jnp.log(h / ph) * 5.0

    locs = locs_ref[0]
    l1 = jnp.zeros((1, p_pad), f32)
    for c, g in enumerate((gx, gy, gw, gh)):
        l1 = l1 + jnp.where(pos, jnp.abs(locs[c:c + 1, :] - g), 0.0)
    loc_sum = jnp.sum(l1, axis=1, keepdims=True)

    sc = scores_ref[0]
    smax = sc[0:1, :]
    for c in range(1, n_cls):
        smax = jnp.maximum(smax, sc[c:c + 1, :])
    sexp = jnp.zeros((1, p_pad), f32)
    for c in range(n_cls):
        sexp = sexp + jnp.exp(sc[c:c + 1, :] - smax)
    lse = smax + jnp.log(sexp)
    s_lab = jnp.where(pos, sc[1:2, :], sc[0:1, :])
    ce = lse - s_lab

    posf = jnp.where(pos, 1.0, 0.0)
    n_pos = jnp.sum(posf, axis=1, keepdims=True)
    conf_pos = jnp.sum(jnp.where(pos, ce, 0.0), axis=1, keepdims=True)
    v = jnp.where(jnp.logical_and(valid, jnp.logical_not(pos)), ce, 0.0)

    k = _NEG_POS_RATIO * n_pos
    lo = jnp.zeros((1, 1), f32)
    hi = jnp.max(v, axis=1, keepdims=True) + 1.0

    def body(_, carry):
        lo_, hi_ = carry
        mid = (lo_ + hi_) * 0.5
        cnt = jnp.sum(jnp.where(v > mid, 1.0, 0.0), axis=1, keepdims=True)
        ge = cnt >= k
        return jnp.where(ge, mid, lo_), jnp.where(ge, hi_, mid)

    lo, hi = jax.lax.fori_loop(0, _BISECT_ITERS, body, (lo, hi))
    cnt_hi = jnp.sum(jnp.where(v > hi, 1.0, 0.0), axis=1, keepdims=True)
    hard = (jnp.sum(jnp.where(v > hi, v, 0.0), axis=1, keepdims=True)
            + jnp.maximum(k - cnt_hi, 0.0) * hi)

    lane128 = jax.lax.broadcasted_iota(jnp.int32, (1, 128), 1)
    out = (jnp.where(lane128 == 0, loc_sum, 0.0)
           + jnp.where(lane128 == 1, conf_pos, 0.0)
           + jnp.where(lane128 == 2, hard, 0.0)
           + jnp.where(lane128 == 3, n_pos, 0.0))
    out_ref[0] = out


def kernel(predicted_locs, predicted_scores, boxes, priors_cxcy):
    B, P, _ = predicted_locs.shape
    n_obj = boxes.shape[1]
    n_cls = predicted_scores.shape[-1]
    p_pad = ((P + 1023) // 1024) * 1024
    pad = p_pad - P

    locs_t = jnp.pad(predicted_locs, ((0, 0), (0, pad), (0, 0))).transpose(0, 2, 1)
    scores_t = jnp.pad(predicted_scores, ((0, 0), (0, pad), (0, 0))).transpose(0, 2, 1)
    priors_t = jnp.pad(priors_cxcy, ((0, pad), (0, 0))).T

    partials = pl.pallas_call(
        functools.partial(_mbl_kernel, p_real=P, n_obj=n_obj, n_cls=n_cls,
                          p_pad=p_pad),
        grid=(B,),
        in_specs=[
            pl.BlockSpec((1, 4, p_pad), lambda b: (b, 0, 0)),
            pl.BlockSpec((1, n_cls, p_pad), lambda b: (b, 0, 0)),
            pl.BlockSpec((1, n_obj, 4), lambda b: (b, 0, 0)),
            pl.BlockSpec((4, p_pad), lambda b: (0, 0)),
        ],
        out_specs=pl.BlockSpec((1, 1, 128), lambda b: (b, 0, 0)),
        out_shape=jax.ShapeDtypeStruct((B, 1, 128), jnp.float32),
    )(locs_t, scores_t, boxes, priors_t)

    loc_sum = jnp.sum(partials[:, 0, 0])
    conf_pos = jnp.sum(partials[:, 0, 1])
    hard = jnp.sum(partials[:, 0, 2])
    n_pos = jnp.sum(partials[:, 0, 3])
    loc_loss = loc_sum / (n_pos * 4.0)
    conf_loss = (hard + conf_pos) / n_pos
    return conf_loss + _ALPHA * loc_loss


# speedup vs baseline: 11.4371x; 2.4614x over previous
import functools

import jax
import jax.numpy as jnp
from jax.experimental import pallas as pl
from jax.experimental.pallas import tpu as pltpu

_THRESHOLD = 0.2
_NEG_POS_RATIO = 3.0
_ALPHA = 1.0
_BISECT_ITERS = 30


def _mbl_kernel(locs_ref, scores_ref, boxes_ref, priors_ref, out_ref, *,
                p_real, n_obj, n_cls, rows):
    f32 = jnp.float32
    shp = (rows, 128)
    idx = (jax.lax.broadcasted_iota(jnp.int32, shp, 0) * 128
           + jax.lax.broadcasted_iota(jnp.int32, shp, 1))
    valid = idx < p_real

    pri = priors_ref[0]
    pcx = pri[0]
    pcy = pri[1]
    pw = pri[2]
    ph = pri[3]
    px0 = pcx - pw / 2.0
    py0 = pcy - ph / 2.0
    px1 = pcx + pw / 2.0
    py1 = pcy + ph / 2.0
    a2 = (px1 - px0) * (py1 - py0)

    boxes = boxes_ref[0]

    m = jnp.full(shp, -1.0, f32)
    mx0 = jnp.zeros(shp, f32)
    my0 = jnp.zeros(shp, f32)
    mx1 = jnp.zeros(shp, f32)
    my1 = jnp.zeros(shp, f32)
    best_prior = []
    for j in range(n_obj):
        bx0 = boxes[j:j + 1, 0:1]
        by0 = boxes[j:j + 1, 1:2]
        bx1 = boxes[j:j + 1, 2:3]
        by1 = boxes[j:j + 1, 3:4]
        lox = jnp.maximum(bx0, px0)
        loy = jnp.maximum(by0, py0)
        hix = jnp.minimum(bx1, px1)
        hiy = jnp.minimum(by1, py1)
        wx = jnp.clip(hix - lox, 0.0, None)
        wy = jnp.clip(hiy - loy, 0.0, None)
        inter = wx * wy
        a1 = (bx1 - bx0) * (by1 - by0)
        union = a1 + a2 - inter
        ov = inter / union
        mj = jnp.max(ov, axis=(0, 1), keepdims=True)
        pj = jnp.min(jnp.where(ov == mj, idx, rows * 128),
                     axis=(0, 1), keepdims=True)
        best_prior.append(pj)
        upd = ov > m
        m = jnp.where(upd, ov, m)
        mx0 = jnp.where(upd, bx0, mx0)
        my0 = jnp.where(upd, by0, my0)
        mx1 = jnp.where(upd, bx1, mx1)
        my1 = jnp.where(upd, by1, my1)

    for j in range(n_obj):
        cond = idx == best_prior[j]
        m = jnp.where(cond, 1.0, m)
        mx0 = jnp.where(cond, boxes[j:j + 1, 0:1], mx0)
        my0 = jnp.where(cond, boxes[j:j + 1, 1:2], my0)
        mx1 = jnp.where(cond, boxes[j:j + 1, 2:3], mx1)
        my1 = jnp.where(cond, boxes[j:j + 1, 3:4], my1)

    pos = jnp.logical_not(m < _THRESHOLD)

    cx = (mx0 + mx1) / 2.0
    cy = (my0 + my1) / 2.0
    w = mx1 - mx0
    h = my1 - my0
    gx = (cx - pcx) / (pw / 10.0)
    gy = (cy - pcy) / (ph / 10.0)
    gw = jnp.log(w / pw) * 5.0
    gh = jnp.log(h / ph) * 5.0

    locs = locs_ref[0]
    l1 = jnp.zeros(shp, f32)
    for c, g in enumerate((gx, gy, gw, gh)):
        l1 = l1 + jnp.where(pos, jnp.abs(locs[c] - g), 0.0)
    loc_sum = jnp.sum(l1, axis=(0, 1), keepdims=True)

    sc = scores_ref[0]
    smax = sc[0]
    for c in range(1, n_cls):
        smax = jnp.maximum(smax, sc[c])
    sexp = jnp.zeros(shp, f32)
    for c in range(n_cls):
        sexp = sexp + jnp.exp(sc[c] - smax)
    lse = smax + jnp.log(sexp)
    s_lab = jnp.where(pos, sc[1], sc[0])
    ce = lse - s_lab

    posf = jnp.where(pos, 1.0, 0.0)
    n_pos = jnp.sum(posf, axis=(0, 1), keepdims=True)
    conf_pos = jnp.sum(jnp.where(pos, ce, 0.0), axis=(0, 1), keepdims=True)
    v = jnp.where(jnp.logical_and(valid, jnp.logical_not(pos)), ce, 0.0)

    k = _NEG_POS_RATIO * n_pos
    lo = jnp.zeros((1, 1), f32)
    hi = jnp.max(v, axis=(0, 1), keepdims=True) + 1.0

    def body(_, carry):
        lo_, hi_ = carry
        mid = (lo_ + hi_) * 0.5
        cnt = jnp.sum(jnp.where(v > mid, 1.0, 0.0), axis=(0, 1), keepdims=True)
        ge = cnt >= k
        return jnp.where(ge, mid, lo_), jnp.where(ge, hi_, mid)

    lo, hi = jax.lax.fori_loop(0, _BISECT_ITERS, body, (lo, hi))
    cnt_hi = jnp.sum(jnp.where(v > hi, 1.0, 0.0), axis=(0, 1), keepdims=True)
    hard = (jnp.sum(jnp.where(v > hi, v, 0.0), axis=(0, 1), keepdims=True)
            + jnp.maximum(k - cnt_hi, 0.0) * hi)

    lane128 = jax.lax.broadcasted_iota(jnp.int32, (1, 128), 1)
    out = (jnp.where(lane128 == 0, loc_sum, 0.0)
           + jnp.where(lane128 == 1, conf_pos, 0.0)
           + jnp.where(lane128 == 2, hard, 0.0)
           + jnp.where(lane128 == 3, n_pos, 0.0))
    out_ref[0] = out


def kernel(predicted_locs, predicted_scores, boxes, priors_cxcy):
    B, P, _ = predicted_locs.shape
    n_obj = boxes.shape[1]
    n_cls = predicted_scores.shape[-1]
    p_pad = ((P + 1023) // 1024) * 1024
    pad = p_pad - P
    rows = p_pad // 128

    locs_t = jnp.pad(predicted_locs, ((0, 0), (0, pad), (0, 0))) \
        .transpose(0, 2, 1).reshape(B, 4, rows, 128)
    scores_t = jnp.pad(predicted_scores, ((0, 0), (0, pad), (0, 0))) \
        .transpose(0, 2, 1).reshape(B, n_cls, rows, 128)
    priors_t = jnp.pad(priors_cxcy, ((0, pad), (0, 0))) \
        .T.reshape(1, 4, rows, 128)

    partials = pl.pallas_call(
        functools.partial(_mbl_kernel, p_real=P, n_obj=n_obj, n_cls=n_cls,
                          rows=rows),
        grid=(B,),
        in_specs=[
            pl.BlockSpec((1, 4, rows, 128), lambda b: (b, 0, 0, 0)),
            pl.BlockSpec((1, n_cls, rows, 128), lambda b: (b, 0, 0, 0)),
            pl.BlockSpec((1, n_obj, 4), lambda b: (b, 0, 0)),
            pl.BlockSpec((1, 4, rows, 128), lambda b: (0, 0, 0, 0)),
        ],
        out_specs=pl.BlockSpec((1, 1, 128), lambda b: (b, 0, 0)),
        out_shape=jax.ShapeDtypeStruct((B, 1, 128), jnp.float32),
        compiler_params=pltpu.CompilerParams(
            dimension_semantics=("parallel",)),
    )(locs_t, scores_t, boxes, priors_t)

    loc_sum = jnp.sum(partials[:, 0, 0])
    conf_pos = jnp.sum(partials[:, 0, 1])
    hard = jnp.sum(partials[:, 0, 2])
    n_pos = jnp.sum(partials[:, 0, 3])
    loc_loss = loc_sum / (n_pos * 4.0)
    conf_loss = (hard + conf_pos) / n_pos
    return conf_loss + _ALPHA * loc_loss


# speedup vs baseline: 22.5422x; 1.9710x over previous
import functools

import jax
import jax.numpy as jnp
from jax.experimental import pallas as pl
from jax.experimental.pallas import tpu as pltpu

_THRESHOLD = 0.2
_NEG_POS_RATIO = 3.0
_ALPHA = 1.0
_SEARCH_ROUNDS = 12


def _mbl_kernel(locs_ref, scores_ref, boxes_ref, priors_ref, out_ref, *,
                p_real, n_obj, n_cls, rows):
    f32 = jnp.float32
    shp = (rows, 128)
    idx = (jax.lax.broadcasted_iota(jnp.int32, shp, 0) * 128
           + jax.lax.broadcasted_iota(jnp.int32, shp, 1))
    valid = idx < p_real

    pri = priors_ref[0]
    pcx = pri[0]
    pcy = pri[1]
    pw = pri[2]
    ph = pri[3]
    px0 = pcx - pw / 2.0
    py0 = pcy - ph / 2.0
    px1 = pcx + pw / 2.0
    py1 = pcy + ph / 2.0
    a2 = (px1 - px0) * (py1 - py0)

    boxes = boxes_ref[0]
    bx0 = boxes[:, 0:1].reshape(n_obj, 1, 1)
    by0 = boxes[:, 1:2].reshape(n_obj, 1, 1)
    bx1 = boxes[:, 2:3].reshape(n_obj, 1, 1)
    by1 = boxes[:, 3:4].reshape(n_obj, 1, 1)

    lox = jnp.maximum(bx0, px0[None])
    loy = jnp.maximum(by0, py0[None])
    hix = jnp.minimum(bx1, px1[None])
    hiy = jnp.minimum(by1, py1[None])
    wx = jnp.clip(hix - lox, 0.0, None)
    wy = jnp.clip(hiy - loy, 0.0, None)
    inter = wx * wy
    a1 = (bx1 - bx0) * (by1 - by0)
    union = a1 + a2[None] - inter
    ov_all = inter / union

    mj = jnp.max(ov_all, axis=(1, 2), keepdims=True)
    pj = jnp.min(jnp.where(ov_all == mj, idx[None], rows * 128),
                 axis=(1, 2), keepdims=True)

    m = jnp.full(shp, -1.0, f32)
    mx0 = jnp.zeros(shp, f32)
    my0 = jnp.zeros(shp, f32)
    mx1 = jnp.zeros(shp, f32)
    my1 = jnp.zeros(shp, f32)
    for j in range(n_obj):
        ov = ov_all[j]
        upd = ov > m
        m = jnp.where(upd, ov, m)
        mx0 = jnp.where(upd, bx0[j], mx0)
        my0 = jnp.where(upd, by0[j], my0)
        mx1 = jnp.where(upd, bx1[j], mx1)
        my1 = jnp.where(upd, by1[j], my1)

    for j in range(n_obj):
        cond = idx == pj[j]
        m = jnp.where(cond, 1.0, m)
        mx0 = jnp.where(cond, bx0[j], mx0)
        my0 = jnp.where(cond, by0[j], my0)
        mx1 = jnp.where(cond, bx1[j], mx1)
        my1 = jnp.where(cond, by1[j], my1)

    pos = jnp.logical_not(m < _THRESHOLD)

    cx = (mx0 + mx1) / 2.0
    cy = (my0 + my1) / 2.0
    w = mx1 - mx0
    h = my1 - my0
    gx = (cx - pcx) / (pw / 10.0)
    gy = (cy - pcy) / (ph / 10.0)
    gw = jnp.log(w / pw) * 5.0
    gh = jnp.log(h / ph) * 5.0

    locs = locs_ref[0]
    l1 = (jnp.abs(locs[0] - gx) + jnp.abs(locs[1] - gy)
          + jnp.abs(locs[2] - gw) + jnp.abs(locs[3] - gh))
    loc_sum = jnp.sum(jnp.where(pos, l1, 0.0), axis=(0, 1), keepdims=True)

    sc = scores_ref[0]
    smax = sc[0]
    for c in range(1, n_cls):
        smax = jnp.maximum(smax, sc[c])
    sexp = jnp.zeros(shp, f32)
    for c in range(n_cls):
        sexp = sexp + jnp.exp(sc[c] - smax)
    lse = smax + jnp.log(sexp)
    s_lab = jnp.where(pos, sc[1], sc[0])
    ce = lse - s_lab

    n_pos = jnp.sum(jnp.where(pos, 1.0, 0.0), axis=(0, 1), keepdims=True)
    conf_pos = jnp.sum(jnp.where(pos, ce, 0.0), axis=(0, 1), keepdims=True)
    v = jnp.where(jnp.logical_and(valid, jnp.logical_not(pos)), ce, 0.0)

    k = _NEG_POS_RATIO * n_pos
    lo = jnp.zeros((1, 1), f32)
    hi = jnp.max(v, axis=(0, 1), keepdims=True) + 1.0

    def body(_, carry):
        lo_, hi_ = carry
        span = hi_ - lo_
        m1 = lo_ + span * 0.25
        m2 = lo_ + span * 0.5
        m3 = lo_ + span * 0.75
        c1 = jnp.sum(jnp.where(v > m1, 1.0, 0.0), axis=(0, 1), keepdims=True)
        c2 = jnp.sum(jnp.where(v > m2, 1.0, 0.0), axis=(0, 1), keepdims=True)
        c3 = jnp.sum(jnp.where(v > m3, 1.0, 0.0), axis=(0, 1), keepdims=True)
        nlo = jnp.where(c3 >= k, m3,
                        jnp.where(c2 >= k, m2,
                                  jnp.where(c1 >= k, m1, lo_)))
        nhi = jnp.where(c1 < k, m1,
                        jnp.where(c2 < k, m2,
                                  jnp.where(c3 < k, m3, hi_)))
        return nlo, nhi

    lo, hi = jax.lax.fori_loop(0, _SEARCH_ROUNDS, body, (lo, hi))
    cnt_hi = jnp.sum(jnp.where(v > hi, 1.0, 0.0), axis=(0, 1), keepdims=True)
    hard = (jnp.sum(jnp.where(v > hi, v, 0.0), axis=(0, 1), keepdims=True)
            + jnp.maximum(k - cnt_hi, 0.0) * hi)

    lane128 = jax.lax.broadcasted_iota(jnp.int32, (1, 128), 1)
    out = (jnp.where(lane128 == 0, loc_sum, 0.0)
           + jnp.where(lane128 == 1, conf_pos, 0.0)
           + jnp.where(lane128 == 2, hard, 0.0)
           + jnp.where(lane128 == 3, n_pos, 0.0))
    out_ref[0] = out


def kernel(predicted_locs, predicted_scores, boxes, priors_cxcy):
    B, P, _ = predicted_locs.shape
    n_obj = boxes.shape[1]
    n_cls = predicted_scores.shape[-1]
    p_pad = ((P + 1023) // 1024) * 1024
    pad = p_pad - P
    rows = p_pad // 128

    locs_t = jnp.pad(predicted_locs, ((0, 0), (0, pad), (0, 0))) \
        .transpose(0, 2, 1).reshape(B, 4, rows, 128)
    scores_t = jnp.pad(predicted_scores, ((0, 0), (0, pad), (0, 0))) \
        .transpose(0, 2, 1).reshape(B, n_cls, rows, 128)
    priors_t = jnp.pad(priors_cxcy, ((0, pad), (0, 0))) \
        .T.reshape(1, 4, rows, 128)

    partials = pl.pallas_call(
        functools.partial(_mbl_kernel, p_real=P, n_obj=n_obj, n_cls=n_cls,
                          rows=rows),
        grid=(B,),
        in_specs=[
            pl.BlockSpec((1, 4, rows, 128), lambda b: (b, 0, 0, 0)),
            pl.BlockSpec((1, n_cls, rows, 128), lambda b: (b, 0, 0, 0)),
            pl.BlockSpec((1, n_obj, 4), lambda b: (b, 0, 0)),
            pl.BlockSpec((1, 4, rows, 128), lambda b: (0, 0, 0, 0)),
        ],
        out_specs=pl.BlockSpec((1, 1, 128), lambda b: (b, 0, 0)),
        out_shape=jax.ShapeDtypeStruct((B, 1, 128), jnp.float32),
        compiler_params=pltpu.CompilerParams(
            dimension_semantics=("parallel",)),
    )(locs_t, scores_t, boxes, priors_t)

    loc_sum = jnp.sum(partials[:, 0, 0])
    conf_pos = jnp.sum(partials[:, 0, 1])
    hard = jnp.sum(partials[:, 0, 2])
    n_pos = jnp.sum(partials[:, 0, 3])
    loc_loss = loc_sum / (n_pos * 4.0)
    conf_loss = (hard + conf_pos) / n_pos
    return conf_loss + _ALPHA * loc_loss


# speedup vs baseline: 24.2637x; 1.0764x over previous
import functools

import jax
import jax.numpy as jnp
from jax.experimental import pallas as pl
from jax.experimental.pallas import tpu as pltpu

_THRESHOLD = 0.2
_NEG_POS_RATIO = 3.0
_ALPHA = 1.0
_SEARCH_ROUNDS = 12


def _mbl_kernel(locs_ref, scores_ref, boxes_ref, priors_ref, out_ref, *,
                p_real, n_obj, n_cls, rows):
    f32 = jnp.float32
    shp = (rows, 128)
    idx = (jax.lax.broadcasted_iota(jnp.int32, shp, 0) * 128
           + jax.lax.broadcasted_iota(jnp.int32, shp, 1))
    valid = idx < p_real

    pri = priors_ref[0]
    pcx = pri[0]
    pcy = pri[1]
    pw = pri[2]
    ph = pri[3]
    px0 = pcx - pw / 2.0
    py0 = pcy - ph / 2.0
    px1 = pcx + pw / 2.0
    py1 = pcy + ph / 2.0
    a2 = (px1 - px0) * (py1 - py0)

    boxes = boxes_ref[0]
    bx0 = boxes[:, 0:1].reshape(n_obj, 1, 1)
    by0 = boxes[:, 1:2].reshape(n_obj, 1, 1)
    bx1 = boxes[:, 2:3].reshape(n_obj, 1, 1)
    by1 = boxes[:, 3:4].reshape(n_obj, 1, 1)

    lox = jnp.maximum(bx0, px0[None])
    loy = jnp.maximum(by0, py0[None])
    hix = jnp.minimum(bx1, px1[None])
    hiy = jnp.minimum(by1, py1[None])
    wx = jnp.clip(hix - lox, 0.0, None)
    wy = jnp.clip(hiy - loy, 0.0, None)
    inter = wx * wy
    a1 = (bx1 - bx0) * (by1 - by0)
    union = a1 + a2[None] - inter
    ov_all = inter / union

    mj = jnp.max(ov_all, axis=(1, 2), keepdims=True)
    pj = jnp.min(jnp.where(ov_all == mj, idx[None], rows * 128),
                 axis=(1, 2), keepdims=True)

    m = jnp.full(shp, -1.0, f32)
    mx0 = jnp.zeros(shp, f32)
    my0 = jnp.zeros(shp, f32)
    mx1 = jnp.zeros(shp, f32)
    my1 = jnp.zeros(shp, f32)
    for j in range(n_obj):
        ov = jnp.where(idx == pj[j], 2.0 + j, ov_all[j])
        upd = ov > m
        m = jnp.where(upd, ov, m)
        mx0 = jnp.where(upd, bx0[j], mx0)
        my0 = jnp.where(upd, by0[j], my0)
        mx1 = jnp.where(upd, bx1[j], mx1)
        my1 = jnp.where(upd, by1[j], my1)

    pos = jnp.logical_not(m < _THRESHOLD)

    cx = (mx0 + mx1) / 2.0
    cy = (my0 + my1) / 2.0
    w = mx1 - mx0
    h = my1 - my0
    gx = (cx - pcx) / (pw / 10.0)
    gy = (cy - pcy) / (ph / 10.0)
    gw = jnp.log(w / pw) * 5.0
    gh = jnp.log(h / ph) * 5.0

    locs = locs_ref[0]
    l1 = (jnp.abs(locs[0] - gx) + jnp.abs(locs[1] - gy)
          + jnp.abs(locs[2] - gw) + jnp.abs(locs[3] - gh))
    loc_sum = jnp.sum(jnp.where(pos, l1, 0.0), axis=(0, 1), keepdims=True)

    sc = scores_ref[0]
    smax = sc[0]
    for c in range(1, n_cls):
        smax = jnp.maximum(smax, sc[c])
    sexp = jnp.zeros(shp, f32)
    for c in range(n_cls):
        sexp = sexp + jnp.exp(sc[c] - smax)
    lse = smax + jnp.log(sexp)
    s_lab = jnp.where(pos, sc[1], sc[0])
    ce = lse - s_lab

    n_pos = jnp.sum(jnp.where(pos, 1.0, 0.0), axis=(0, 1), keepdims=True)
    conf_pos = jnp.sum(jnp.where(pos, ce, 0.0), axis=(0, 1), keepdims=True)
    v = jnp.where(jnp.logical_and(valid, jnp.logical_not(pos)), ce, 0.0)

    k = _NEG_POS_RATIO * n_pos
    lo = jnp.zeros((1, 1), f32)
    hi = jnp.max(v, axis=(0, 1), keepdims=True) + 1.0

    def body(_, carry):
        lo_, hi_ = carry
        span = hi_ - lo_
        m1 = lo_ + span * 0.25
        m2 = lo_ + span * 0.5
        m3 = lo_ + span * 0.75
        c1 = jnp.sum(jnp.where(v > m1, 1.0, 0.0), axis=(0, 1), keepdims=True)
        c2 = jnp.sum(jnp.where(v > m2, 1.0, 0.0), axis=(0, 1), keepdims=True)
        c3 = jnp.sum(jnp.where(v > m3, 1.0, 0.0), axis=(0, 1), keepdims=True)
        nlo = jnp.where(c3 >= k, m3,
                        jnp.where(c2 >= k, m2,
                                  jnp.where(c1 >= k, m1, lo_)))
        nhi = jnp.where(c1 < k, m1,
                        jnp.where(c2 < k, m2,
                                  jnp.where(c3 < k, m3, hi_)))
        return nlo, nhi

    lo, hi = jax.lax.fori_loop(0, _SEARCH_ROUNDS, body, (lo, hi))
    cnt_hi = jnp.sum(jnp.where(v > hi, 1.0, 0.0), axis=(0, 1), keepdims=True)
    hard = (jnp.sum(jnp.where(v > hi, v, 0.0), axis=(0, 1), keepdims=True)
            + jnp.maximum(k - cnt_hi, 0.0) * hi)

    lane128 = jax.lax.broadcasted_iota(jnp.int32, (1, 128), 1)
    out = (jnp.where(lane128 == 0, loc_sum, 0.0)
           + jnp.where(lane128 == 1, conf_pos, 0.0)
           + jnp.where(lane128 == 2, hard, 0.0)
           + jnp.where(lane128 == 3, n_pos, 0.0))
    out_ref[0] = out


def kernel(predicted_locs, predicted_scores, boxes, priors_cxcy):
    B, P, _ = predicted_locs.shape
    n_obj = boxes.shape[1]
    n_cls = predicted_scores.shape[-1]
    p_pad = ((P + 1023) // 1024) * 1024
    pad = p_pad - P
    rows = p_pad // 128

    locs_t = jnp.pad(predicted_locs, ((0, 0), (0, pad), (0, 0))) \
        .transpose(0, 2, 1).reshape(B, 4, rows, 128)
    scores_t = jnp.pad(predicted_scores, ((0, 0), (0, pad), (0, 0))) \
        .transpose(0, 2, 1).reshape(B, n_cls, rows, 128)
    priors_t = jnp.pad(priors_cxcy, ((0, pad), (0, 0))) \
        .T.reshape(1, 4, rows, 128)

    partials = pl.pallas_call(
        functools.partial(_mbl_kernel, p_real=P, n_obj=n_obj, n_cls=n_cls,
                          rows=rows),
        grid=(B,),
        in_specs=[
            pl.BlockSpec((1, 4, rows, 128), lambda b: (b, 0, 0, 0)),
            pl.BlockSpec((1, n_cls, rows, 128), lambda b: (b, 0, 0, 0)),
            pl.BlockSpec((1, n_obj, 4), lambda b: (b, 0, 0)),
            pl.BlockSpec((1, 4, rows, 128), lambda b: (0, 0, 0, 0)),
        ],
        out_specs=pl.BlockSpec((1, 1, 128), lambda b: (b, 0, 0)),
        out_shape=jax.ShapeDtypeStruct((B, 1, 128), jnp.float32),
        compiler_params=pltpu.CompilerParams(
            dimension_semantics=("parallel",)),
    )(locs_t, scores_t, boxes, priors_t)

    loc_sum = jnp.sum(partials[:, 0, 0])
    conf_pos = jnp.sum(partials[:, 0, 1])
    hard = jnp.sum(partials[:, 0, 2])
    n_pos = jnp.sum(partials[:, 0, 3])
    loc_loss = loc_sum / (n_pos * 4.0)
    conf_loss = (hard + conf_pos) / n_pos
    return conf_loss + _ALPHA * loc_loss


# speedup vs baseline: 24.5317x; 1.0110x over previous
import functools

import jax
import jax.numpy as jnp
from jax.experimental import pallas as pl
from jax.experimental.pallas import tpu as pltpu

_THRESHOLD = 0.2
_NEG_POS_RATIO = 3.0
_ALPHA = 1.0
_SEARCH_ROUNDS = 12


def _mbl_kernel(locs_ref, scores_ref, boxes_ref, priors_ref, out_ref, *,
                p_real, n_obj, n_cls, rows, rpp):
    for _i in range(rpp):
        _mbl_row(locs_ref, scores_ref, boxes_ref, priors_ref, out_ref, _i,
                 p_real=p_real, n_obj=n_obj, n_cls=n_cls, rows=rows)


def _mbl_row(locs_ref, scores_ref, boxes_ref, priors_ref, out_ref, i, *,
             p_real, n_obj, n_cls, rows):
    f32 = jnp.float32
    shp = (rows, 128)
    idx = (jax.lax.broadcasted_iota(jnp.int32, shp, 0) * 128
           + jax.lax.broadcasted_iota(jnp.int32, shp, 1))
    valid = idx < p_real

    pri = priors_ref[0]
    pcx = pri[0]
    pcy = pri[1]
    pw = pri[2]
    ph = pri[3]
    px0 = pcx - pw / 2.0
    py0 = pcy - ph / 2.0
    px1 = pcx + pw / 2.0
    py1 = pcy + ph / 2.0
    a2 = (px1 - px0) * (py1 - py0)

    boxes = boxes_ref[i]
    bx0 = boxes[:, 0:1].reshape(n_obj, 1, 1)
    by0 = boxes[:, 1:2].reshape(n_obj, 1, 1)
    bx1 = boxes[:, 2:3].reshape(n_obj, 1, 1)
    by1 = boxes[:, 3:4].reshape(n_obj, 1, 1)

    lox = jnp.maximum(bx0, px0[None])
    loy = jnp.maximum(by0, py0[None])
    hix = jnp.minimum(bx1, px1[None])
    hiy = jnp.minimum(by1, py1[None])
    wx = jnp.clip(hix - lox, 0.0, None)
    wy = jnp.clip(hiy - loy, 0.0, None)
    inter = wx * wy
    a1 = (bx1 - bx0) * (by1 - by0)
    union = a1 + a2[None] - inter
    ov_all = inter / union

    mj = jnp.max(ov_all, axis=(1, 2), keepdims=True)
    pj = jnp.min(jnp.where(ov_all == mj, idx[None], rows * 128),
                 axis=(1, 2), keepdims=True)

    m = jnp.full(shp, -1.0, f32)
    mx0 = jnp.zeros(shp, f32)
    my0 = jnp.zeros(shp, f32)
    mx1 = jnp.zeros(shp, f32)
    my1 = jnp.zeros(shp, f32)
    for j in range(n_obj):
        ov = jnp.where(idx == pj[j], 2.0 + j, ov_all[j])
        upd = ov > m
        m = jnp.where(upd, ov, m)
        mx0 = jnp.where(upd, bx0[j], mx0)
        my0 = jnp.where(upd, by0[j], my0)
        mx1 = jnp.where(upd, bx1[j], mx1)
        my1 = jnp.where(upd, by1[j], my1)

    pos = jnp.logical_not(m < _THRESHOLD)

    cx = (mx0 + mx1) / 2.0
    cy = (my0 + my1) / 2.0
    w = mx1 - mx0
    h = my1 - my0
    gx = (cx - pcx) / (pw / 10.0)
    gy = (cy - pcy) / (ph / 10.0)
    gw = jnp.log(w / pw) * 5.0
    gh = jnp.log(h / ph) * 5.0

    locs = locs_ref[i]
    l1 = (jnp.abs(locs[0] - gx) + jnp.abs(locs[1] - gy)
          + jnp.abs(locs[2] - gw) + jnp.abs(locs[3] - gh))
    loc_sum = jnp.sum(jnp.where(pos, l1, 0.0), axis=(0, 1), keepdims=True)

    sc = scores_ref[i]
    smax = sc[0]
    for c in range(1, n_cls):
        smax = jnp.maximum(smax, sc[c])
    sexp = jnp.zeros(shp, f32)
    for c in range(n_cls):
        sexp = sexp + jnp.exp(sc[c] - smax)
    lse = smax + jnp.log(sexp)
    s_lab = jnp.where(pos, sc[1], sc[0])
    ce = lse - s_lab

    n_pos = jnp.sum(jnp.where(pos, 1.0, 0.0), axis=(0, 1), keepdims=True)
    conf_pos = jnp.sum(jnp.where(pos, ce, 0.0), axis=(0, 1), keepdims=True)
    v = jnp.where(jnp.logical_and(valid, jnp.logical_not(pos)), ce, 0.0)

    k = _NEG_POS_RATIO * n_pos
    lo = jnp.zeros((1, 1), f32)
    hi = jnp.max(v, axis=(0, 1), keepdims=True) + 1.0

    def body(_, carry):
        lo_, hi_ = carry
        span = hi_ - lo_
        m1 = lo_ + span * 0.25
        m2 = lo_ + span * 0.5
        m3 = lo_ + span * 0.75
        c1 = jnp.sum(jnp.where(v > m1, 1.0, 0.0), axis=(0, 1), keepdims=True)
        c2 = jnp.sum(jnp.where(v > m2, 1.0, 0.0), axis=(0, 1), keepdims=True)
        c3 = jnp.sum(jnp.where(v > m3, 1.0, 0.0), axis=(0, 1), keepdims=True)
        nlo = jnp.where(c3 >= k, m3,
                        jnp.where(c2 >= k, m2,
                                  jnp.where(c1 >= k, m1, lo_)))
        nhi = jnp.where(c1 < k, m1,
                        jnp.where(c2 < k, m2,
                                  jnp.where(c3 < k, m3, hi_)))
        return nlo, nhi

    lo, hi = jax.lax.fori_loop(0, _SEARCH_ROUNDS, body, (lo, hi))
    cnt_hi = jnp.sum(jnp.where(v > hi, 1.0, 0.0), axis=(0, 1), keepdims=True)
    hard = (jnp.sum(jnp.where(v > hi, v, 0.0), axis=(0, 1), keepdims=True)
            + jnp.maximum(k - cnt_hi, 0.0) * hi)

    lane128 = jax.lax.broadcasted_iota(jnp.int32, (1, 128), 1)
    out = (jnp.where(lane128 == 0, loc_sum, 0.0)
           + jnp.where(lane128 == 1, conf_pos, 0.0)
           + jnp.where(lane128 == 2, hard, 0.0)
           + jnp.where(lane128 == 3, n_pos, 0.0))
    out_ref[i] = out


def kernel(predicted_locs, predicted_scores, boxes, priors_cxcy):
    B, P, _ = predicted_locs.shape
    n_obj = boxes.shape[1]
    n_cls = predicted_scores.shape[-1]
    p_pad = ((P + 1023) // 1024) * 1024
    pad = p_pad - P
    rows = p_pad // 128

    locs_t = jnp.pad(predicted_locs, ((0, 0), (0, pad), (0, 0))) \
        .transpose(0, 2, 1).reshape(B, 4, rows, 128)
    scores_t = jnp.pad(predicted_scores, ((0, 0), (0, pad), (0, 0))) \
        .transpose(0, 2, 1).reshape(B, n_cls, rows, 128)
    priors_t = jnp.pad(priors_cxcy, ((0, pad), (0, 0))) \
        .T.reshape(1, 4, rows, 128)

    rpp = 2 if B % 2 == 0 else 1
    partials = pl.pallas_call(
        functools.partial(_mbl_kernel, p_real=P, n_obj=n_obj, n_cls=n_cls,
                          rows=rows, rpp=rpp),
        grid=(B // rpp,),
        in_specs=[
            pl.BlockSpec((rpp, 4, rows, 128), lambda b: (b, 0, 0, 0)),
            pl.BlockSpec((rpp, n_cls, rows, 128), lambda b: (b, 0, 0, 0)),
            pl.BlockSpec((rpp, n_obj, 4), lambda b: (b, 0, 0)),
            pl.BlockSpec((1, 4, rows, 128), lambda b: (0, 0, 0, 0)),
        ],
        out_specs=pl.BlockSpec((rpp, 1, 128), lambda b: (b, 0, 0)),
        out_shape=jax.ShapeDtypeStruct((B, 1, 128), jnp.float32),
        compiler_params=pltpu.CompilerParams(
            dimension_semantics=("parallel",)),
    )(locs_t, scores_t, boxes, priors_t)

    loc_sum = jnp.sum(partials[:, 0, 0])
    conf_pos = jnp.sum(partials[:, 0, 1])
    hard = jnp.sum(partials[:, 0, 2])
    n_pos = jnp.sum(partials[:, 0, 3])
    loc_loss = loc_sum / (n_pos * 4.0)
    conf_loss = (hard + conf_pos) / n_pos
    return conf_loss + _ALPHA * loc_loss


# speedup vs baseline: 24.7162x; 1.0075x over previous
import functools

import jax
import jax.numpy as jnp
from jax.experimental import pallas as pl
from jax.experimental.pallas import tpu as pltpu

_THRESHOLD = 0.2
_NEG_POS_RATIO = 3.0
_ALPHA = 1.0
_SEARCH_ROUNDS = 12


def _mbl_kernel(locs_ref, scores_ref, boxes_ref, priors_ref, out_ref, *,
                p_real, n_obj, n_cls, rows, rpp):
    for _i in range(rpp):
        _mbl_row(locs_ref, scores_ref, boxes_ref, priors_ref, out_ref, _i,
                 p_real=p_real, n_obj=n_obj, n_cls=n_cls, rows=rows)


def _mbl_row(locs_ref, scores_ref, boxes_ref, priors_ref, out_ref, i, *,
             p_real, n_obj, n_cls, rows):
    f32 = jnp.float32
    shp = (rows, 128)
    idx = (jax.lax.broadcasted_iota(jnp.int32, shp, 0) * 128
           + jax.lax.broadcasted_iota(jnp.int32, shp, 1))
    valid = idx < p_real

    pri = priors_ref[0]
    pcx = pri[0]
    pcy = pri[1]
    pw = pri[2]
    ph = pri[3]
    px0 = pcx - pw / 2.0
    py0 = pcy - ph / 2.0
    px1 = pcx + pw / 2.0
    py1 = pcy + ph / 2.0
    a2 = (px1 - px0) * (py1 - py0)

    boxes = boxes_ref[i]
    bx0 = boxes[:, 0:1].reshape(n_obj, 1, 1)
    by0 = boxes[:, 1:2].reshape(n_obj, 1, 1)
    bx1 = boxes[:, 2:3].reshape(n_obj, 1, 1)
    by1 = boxes[:, 3:4].reshape(n_obj, 1, 1)

    lox = jnp.maximum(bx0, px0[None])
    loy = jnp.maximum(by0, py0[None])
    hix = jnp.minimum(bx1, px1[None])
    hiy = jnp.minimum(by1, py1[None])
    wx = jnp.clip(hix - lox, 0.0, None)
    wy = jnp.clip(hiy - loy, 0.0, None)
    inter = wx * wy
    a1 = (bx1 - bx0) * (by1 - by0)
    union = a1 + a2[None] - inter
    ov_all = inter / union

    mj = jnp.max(ov_all, axis=(1, 2), keepdims=True)
    pj = jnp.min(jnp.where(ov_all == mj, idx[None], rows * 128),
                 axis=(1, 2), keepdims=True)

    m = jnp.full(shp, -1.0, f32)
    mx0 = jnp.zeros(shp, f32)
    my0 = jnp.zeros(shp, f32)
    mx1 = jnp.zeros(shp, f32)
    my1 = jnp.zeros(shp, f32)
    for j in range(n_obj):
        ov = jnp.where(idx == pj[j], 2.0 + j, ov_all[j])
        upd = ov > m
        m = jnp.where(upd, ov, m)
        mx0 = jnp.where(upd, bx0[j], mx0)
        my0 = jnp.where(upd, by0[j], my0)
        mx1 = jnp.where(upd, bx1[j], mx1)
        my1 = jnp.where(upd, by1[j], my1)

    pos = jnp.logical_not(m < _THRESHOLD)

    cx = (mx0 + mx1) / 2.0
    cy = (my0 + my1) / 2.0
    w = mx1 - mx0
    h = my1 - my0
    gx = (cx - pcx) / (pw / 10.0)
    gy = (cy - pcy) / (ph / 10.0)
    gw = jnp.log(w / pw) * 5.0
    gh = jnp.log(h / ph) * 5.0

    locs = locs_ref[i]
    l1 = (jnp.abs(locs[0] - gx) + jnp.abs(locs[1] - gy)
          + jnp.abs(locs[2] - gw) + jnp.abs(locs[3] - gh))
    loc_sum = jnp.sum(jnp.where(pos, l1, 0.0), axis=(0, 1), keepdims=True)

    sc = scores_ref[i]
    smax = sc[0]
    for c in range(1, n_cls):
        smax = jnp.maximum(smax, sc[c])
    sexp = jnp.zeros(shp, f32)
    for c in range(n_cls):
        sexp = sexp + jnp.exp(sc[c] - smax)
    lse = smax + jnp.log(sexp)
    s_lab = jnp.where(pos, sc[1], sc[0])
    ce = lse - s_lab

    n_pos = jnp.sum(jnp.where(pos, 1.0, 0.0), axis=(0, 1), keepdims=True)
    conf_pos = jnp.sum(jnp.where(pos, ce, 0.0), axis=(0, 1), keepdims=True)
    v = jnp.where(jnp.logical_and(valid, jnp.logical_not(pos)), ce, 0.0)

    k = _NEG_POS_RATIO * n_pos
    lo = jnp.zeros((1, 1), f32)
    hi = jnp.max(v, axis=(0, 1), keepdims=True) + 1.0

    def body(_, carry):
        lo_, hi_ = carry
        span = hi_ - lo_
        m1 = lo_ + span * 0.25
        m2 = lo_ + span * 0.5
        m3 = lo_ + span * 0.75
        c1 = jnp.sum(jnp.where(v > m1, 1.0, 0.0), axis=(0, 1), keepdims=True)
        c2 = jnp.sum(jnp.where(v > m2, 1.0, 0.0), axis=(0, 1), keepdims=True)
        c3 = jnp.sum(jnp.where(v > m3, 1.0, 0.0), axis=(0, 1), keepdims=True)
        nlo = jnp.where(c3 >= k, m3,
                        jnp.where(c2 >= k, m2,
                                  jnp.where(c1 >= k, m1, lo_)))
        nhi = jnp.where(c1 < k, m1,
                        jnp.where(c2 < k, m2,
                                  jnp.where(c3 < k, m3, hi_)))
        return nlo, nhi

    lo, hi = jax.lax.fori_loop(0, _SEARCH_ROUNDS, body, (lo, hi))
    cnt_hi = jnp.sum(jnp.where(v > hi, 1.0, 0.0), axis=(0, 1), keepdims=True)
    hard = (jnp.sum(jnp.where(v > hi, v, 0.0), axis=(0, 1), keepdims=True)
            + jnp.maximum(k - cnt_hi, 0.0) * hi)

    lane128 = jax.lax.broadcasted_iota(jnp.int32, (1, 128), 1)
    out = (jnp.where(lane128 == 0, loc_sum, 0.0)
           + jnp.where(lane128 == 1, conf_pos, 0.0)
           + jnp.where(lane128 == 2, hard, 0.0)
           + jnp.where(lane128 == 3, n_pos, 0.0))
    out_ref[i] = out


def kernel(predicted_locs, predicted_scores, boxes, priors_cxcy):
    B, P, _ = predicted_locs.shape
    n_obj = boxes.shape[1]
    n_cls = predicted_scores.shape[-1]
    p_pad = ((P + 1023) // 1024) * 1024
    pad = p_pad - P
    rows = p_pad // 128

    locs_t = jnp.pad(predicted_locs, ((0, 0), (0, pad), (0, 0))) \
        .transpose(0, 2, 1).reshape(B, 4, rows, 128)
    scores_t = jnp.pad(predicted_scores, ((0, 0), (0, pad), (0, 0))) \
        .transpose(0, 2, 1).reshape(B, n_cls, rows, 128)
    priors_t = jnp.pad(priors_cxcy, ((0, pad), (0, 0))) \
        .T.reshape(1, 4, rows, 128)

    rpp = 4 if B % 4 == 0 else (2 if B % 2 == 0 else 1)
    partials = pl.pallas_call(
        functools.partial(_mbl_kernel, p_real=P, n_obj=n_obj, n_cls=n_cls,
                          rows=rows, rpp=rpp),
        grid=(B // rpp,),
        in_specs=[
            pl.BlockSpec((rpp, 4, rows, 128), lambda b: (b, 0, 0, 0)),
            pl.BlockSpec((rpp, n_cls, rows, 128), lambda b: (b, 0, 0, 0)),
            pl.BlockSpec((rpp, n_obj, 4), lambda b: (b, 0, 0)),
            pl.BlockSpec((1, 4, rows, 128), lambda b: (0, 0, 0, 0)),
        ],
        out_specs=pl.BlockSpec((rpp, 1, 128), lambda b: (b, 0, 0)),
        out_shape=jax.ShapeDtypeStruct((B, 1, 128), jnp.float32),
        compiler_params=pltpu.CompilerParams(
            dimension_semantics=("parallel",)),
    )(locs_t, scores_t, boxes, priors_t)

    loc_sum = jnp.sum(partials[:, 0, 0])
    conf_pos = jnp.sum(partials[:, 0, 1])
    hard = jnp.sum(partials[:, 0, 2])
    n_pos = jnp.sum(partials[:, 0, 3])
    loc_loss = loc_sum / (n_pos * 4.0)
    conf_loss = (hard + conf_pos) / n_pos
    return conf_loss + _ALPHA * loc_loss
